# baseline (speedup 1.0000x reference)
import jax
import jax.numpy as jnp
from jax.experimental import pallas as pl
from jax.experimental.pallas import tpu as pltpu

_CHUNK_ROWS = 512
_NBUF = 6


def _add_body(idx_ref, x_hbm, emb_ref, o_hbm, buf, sin, sout):
    n_chunks = x_hbm.shape[0] // _CHUNK_ROWS
    row = emb_ref[idx_ref[0]]

    def in_c(g):
        return pltpu.make_async_copy(
            x_hbm.at[pl.ds(g * _CHUNK_ROWS, _CHUNK_ROWS)],
            buf.at[g % _NBUF],
            sin.at[g % _NBUF],
        )

    def out_c(g):
        return pltpu.make_async_copy(
            buf.at[g % _NBUF],
            o_hbm.at[pl.ds(g * _CHUNK_ROWS, _CHUNK_ROWS)],
            sout.at[g % _NBUF],
        )

    in_c(0).start()
    if n_chunks > 1:
        in_c(1).start()
    for g in range(n_chunks):
        if g + 2 < n_chunks:
            if g >= 2:
                out_c(g - 2).wait()
            in_c(g + 2).start()
        in_c(g).wait()
        buf[g % _NBUF] = buf[g % _NBUF] + row
        out_c(g).start()
    for g in range(max(0, n_chunks - 4), n_chunks):
        out_c(g).wait()


def kernel(x, layer_idx, step_embeddings):
    B, S, D = x.shape
    rows = B * S
    x2 = x.reshape(rows, D)
    n_table = step_embeddings.shape[0]
    idx = jnp.asarray(layer_idx, dtype=jnp.int32).reshape(1)
    out = pl.pallas_call(
        _add_body,
        in_specs=[
            pl.BlockSpec(memory_space=pltpu.SMEM),
            pl.BlockSpec(memory_space=pl.ANY),
            pl.BlockSpec(memory_space=pltpu.VMEM),
        ],
        out_specs=pl.BlockSpec(memory_space=pl.ANY),
        out_shape=jax.ShapeDtypeStruct((rows, D), x.dtype),
        scratch_shapes=[
            pltpu.VMEM((_NBUF, _CHUNK_ROWS, D), jnp.float32),
            pltpu.SemaphoreType.DMA((_NBUF,)),
            pltpu.SemaphoreType.DMA((_NBUF,)),
        ],
    )(idx, x2, step_embeddings)
    return out.reshape(B, S, D)


# speedup vs baseline: 1.0085x; 1.0085x over previous
import jax
import jax.numpy as jnp
from jax.experimental import pallas as pl
from jax.experimental.pallas import tpu as pltpu

_CHUNK_ROWS = 2048
_NBUF = 3


def _add_body(idx_ref, x_hbm, emb_ref, o_hbm, buf, sin, sout):
    n_chunks = x_hbm.shape[0] // _CHUNK_ROWS
    row = emb_ref[idx_ref[0]]

    def in_c(g):
        return pltpu.make_async_copy(
            x_hbm.at[pl.ds(g * _CHUNK_ROWS, _CHUNK_ROWS)],
            buf.at[g % _NBUF],
            sin.at[g % _NBUF],
        )

    def out_c(g):
        return pltpu.make_async_copy(
            buf.at[g % _NBUF],
            o_hbm.at[pl.ds(g * _CHUNK_ROWS, _CHUNK_ROWS)],
            sout.at[g % _NBUF],
        )

    dist = _NBUF - 2
    for k in range(min(dist, n_chunks)):
        in_c(k).start()
    for g in range(n_chunks):
        nxt = g + dist
        if nxt < n_chunks:
            if nxt - _NBUF >= 0:
                out_c(nxt - _NBUF).wait()
            in_c(nxt).start()
        in_c(g).wait()
        buf[g % _NBUF] = buf[g % _NBUF] + row
        out_c(g).start()
    for g in range(max(0, n_chunks - _NBUF), n_chunks):
        out_c(g).wait()


def kernel(x, layer_idx, step_embeddings):
    B, S, D = x.shape
    rows = B * S
    x2 = x.reshape(rows, D)
    n_table = step_embeddings.shape[0]
    idx = jnp.asarray(layer_idx, dtype=jnp.int32).reshape(1)
    out = pl.pallas_call(
        _add_body,
        in_specs=[
            pl.BlockSpec(memory_space=pltpu.SMEM),
            pl.BlockSpec(memory_space=pl.ANY),
            pl.BlockSpec(memory_space=pltpu.VMEM),
        ],
        out_specs=pl.BlockSpec(memory_space=pl.ANY),
        out_shape=jax.ShapeDtypeStruct((rows, D), x.dtype),
        scratch_shapes=[
            pltpu.VMEM((_NBUF, _CHUNK_ROWS, D), jnp.float32),
            pltpu.SemaphoreType.DMA((_NBUF,)),
            pltpu.SemaphoreType.DMA((_NBUF,)),
        ],
    )(idx, x2, step_embeddings)
    return out.reshape(B, S, D)


# speedup vs baseline: 1.0129x; 1.0044x over previous
import jax
import jax.numpy as jnp
from jax.experimental import pallas as pl
from jax.experimental.pallas import tpu as pltpu

_BLOCK_ROWS = 1024


def _add_body(idx_ref, x_ref, emb_ref, o_ref):
    row = emb_ref[idx_ref[0]]
    o_ref[...] = x_ref[...] + row


def kernel(x, layer_idx, step_embeddings):
    B, S, D = x.shape
    rows = B * S
    x2 = x.reshape(rows, D)
    n_table = step_embeddings.shape[0]
    block = min(_BLOCK_ROWS, rows)
    grid = rows // block
    idx = jnp.asarray(layer_idx, dtype=jnp.int32).reshape(1)
    out = pl.pallas_call(
        _add_body,
        grid=(grid,),
        in_specs=[
            pl.BlockSpec(memory_space=pltpu.SMEM),
            pl.BlockSpec((block, D), lambda i: (i, 0)),
            pl.BlockSpec((n_table, D), lambda i: (0, 0)),
        ],
        out_specs=pl.BlockSpec((block, D), lambda i: (i, 0)),
        out_shape=jax.ShapeDtypeStruct((rows, D), x.dtype),
        compiler_params=pltpu.CompilerParams(
            dimension_semantics=("parallel",),
        ),
    )(idx, x2, step_embeddings)
    return out.reshape(B, S, D)
